# baseline (speedup 1.0000x reference)
import jax
import jax.numpy as jnp
from jax import lax
from jax.experimental import pallas as pl
from jax.experimental.pallas import tpu as pltpu
from jax.experimental.pallas import tpu_sc as plsc

MIN_KEPT = 100000
L07_BITS = 0x3EB69E19
C = 19
B = 4
H = 512
W = 512
N = B * H * W
ROWS = 64

NT = 16
CHUNK = N // NT
NVREG = CHUNK // 16
NB = 1024
KSEL = N - MIN_KEPT


def _ce_block(out_ref, tgt_ref, loss_ref):
    x = out_ref[0]
    t = tgt_ref[0]
    m = jnp.max(x, axis=0)
    e = jnp.exp(x - m[None])
    s = jnp.sum(e, axis=0)
    cls = lax.broadcasted_iota(jnp.int32, (C, ROWS, W), 0)
    xt = jnp.sum(jnp.where(cls == t[None], x, 0.0), axis=0)
    loss_ref[0] = jnp.log(s) - (xt - m)


def _ce_stage(outputs, target):
    grid = (B, H // ROWS)
    return pl.pallas_call(
        _ce_block,
        grid=grid,
        in_specs=[
            pl.BlockSpec((1, C, ROWS, W), lambda b, r: (b, 0, r, 0)),
            pl.BlockSpec((1, ROWS, W), lambda b, r: (b, r, 0)),
        ],
        out_specs=pl.BlockSpec((1, ROWS, W), lambda b, r: (b, r, 0)),
        out_shape=jax.ShapeDtypeStruct((B, H, W), jnp.float32),
    )(outputs, target)


def _sc_select_body(loss_hbm, out_hbm,
                    loss_v, hist_v, lred_v, hs_v, sums_v, su_v, out_v,
                    hist_sh, sums_sh, semp):
    w = lax.axis_index("s")
    lane = lax.broadcasted_iota(jnp.int32, (16,), 0)
    ones = jnp.full((16,), 1, jnp.int32)
    zeros16 = jnp.zeros((16,), jnp.int32)

    with jax.named_scope("sc_stage_dma"):
        pltpu.async_copy(
            loss_hbm.at[pl.ds(w * CHUNK, CHUNK)], loss_v, semp).wait()

    prefix = jnp.int32(0)
    kkeep = jnp.int32(KSEL)

    for sh_part, sh_buck in [(30, 20), (20, 10), (10, 0)]:
        def hz(j, _):
            for u in range(8):
                hist_v[pl.ds((j * 8 + u) * 16, 16)] = zeros16
            return 0
        with jax.named_scope("sc_hz"):
            lax.fori_loop(0, NB * 16 // 128, hz, 0)

        pref_hi = prefix >> sh_part

        def hist_step(j, _):
            for u in range(8):
                bits = plsc.bitcast(
                    loss_v[pl.ds((j * 8 + u) * 16, 16)], jnp.int32)
                part = (bits >> sh_part) == pref_hi
                bucket = (bits >> sh_buck) & (NB - 1)
                plsc.addupdate_scatter(
                    hist_v, [lane * NB + bucket], ones, mask=part)
            return 0
        with jax.named_scope("sc_hist"):
            lax.fori_loop(0, NVREG // 8, hist_step, 0)

        def lred_step(j, _):
            for u in range(4):
                acc = hist_v[pl.ds((j * 4 + u) * 16, 16)]
                for l in range(1, 16):
                    acc = acc + hist_v[pl.ds(l * NB + (j * 4 + u) * 16, 16)]
                lred_v[pl.ds((j * 4 + u) * 16, 16)] = acc
            return 0
        with jax.named_scope("sc_lred"):
            lax.fori_loop(0, 16, lred_step, 0)

        with jax.named_scope("sc_slots"):
            pltpu.sync_copy(lred_v, hist_sh.at[pl.ds(w * NB, NB)])
            plsc.subcore_barrier()

            pltpu.sync_copy(hist_sh, hs_v)

        def scan_step(j, carry):
            tot, accb, accc = carry
            h = hs_v[pl.ds(j * 16, 16)]
            for t in range(1, NT):
                h = h + hs_v[pl.ds(t * NB + j * 16, 16)]
            c = plsc.cumsum(h) + tot
            m = c < kkeep
            accb = accb + jnp.where(m, 1, 0)
            accc = accc + jnp.where(m, h, 0)
            return tot + jnp.sum(h), accb, accc
        with jax.named_scope("sc_scan"):
            _, accb, accc = lax.fori_loop(
                0, 64, scan_step, (jnp.int32(0), zeros16, zeros16))
        b_star = jnp.sum(accb)
        cnt_before = jnp.sum(accc)
        prefix = prefix | (b_star << sh_buck)
        kkeep = kkeep - cnt_before
        plsc.subcore_barrier()

    thr = jnp.minimum(prefix, jnp.int32(L07_BITS))

    def fin_step(j, carry):
        sa, ca = carry
        for u in range(8):
            lo = loss_v[pl.ds((j * 8 + u) * 16, 16)]
            m = plsc.bitcast(lo, jnp.int32) > thr
            sa = sa + jnp.where(m, lo, 0.0)
            ca = ca + jnp.where(m, 1.0, 0.0)
        return sa, ca
    with jax.named_scope("sc_final"):
        s_acc, c_acc = lax.fori_loop(
            0, NVREG // 8, fin_step,
            (jnp.zeros((16,), jnp.float32), jnp.zeros((16,), jnp.float32)))

    sums_v[pl.ds(0, 16)] = s_acc
    sums_v[pl.ds(16, 16)] = c_acc
    pltpu.sync_copy(sums_v, sums_sh.at[pl.ds(32 * w, 32)])
    plsc.subcore_barrier()

    @pl.when(w == 0)
    def _():
        pltpu.sync_copy(sums_sh, su_v)

        def red_step(t, carry):
            sv, cv = carry
            return (sv + su_v[pl.ds(t * 32, 16)],
                    cv + su_v[pl.ds(t * 32 + 16, 16)])
        sv, cv = lax.fori_loop(
            0, NT, red_step,
            (jnp.zeros((16,), jnp.float32), jnp.zeros((16,), jnp.float32)))
        tvec = jnp.full((16,), jnp.sum(sv), jnp.float32)
        dvec = jnp.maximum(jnp.full((16,), jnp.sum(cv), jnp.float32), 1.0)
        out_v[...] = tvec / dvec
        pltpu.sync_copy(out_v, out_hbm)


def _select_stage(loss):
    mesh = plsc.VectorSubcoreMesh(
        core_axis_name="c", subcore_axis_name="s", num_cores=1)
    f = pl.kernel(
        _sc_select_body,
        out_type=jax.ShapeDtypeStruct((16,), jnp.float32),
        mesh=mesh,
        compiler_params=pltpu.CompilerParams(needs_layout_passes=False),
        scratch_types=[
            pltpu.VMEM((CHUNK,), jnp.float32),
            pltpu.VMEM((16 * NB,), jnp.int32),
            pltpu.VMEM((NB,), jnp.int32),
            pltpu.VMEM((NT * NB,), jnp.int32),
            pltpu.VMEM((32,), jnp.float32),
            pltpu.VMEM((NT * 32,), jnp.float32),
            pltpu.VMEM((16,), jnp.float32),
            pltpu.VMEM_SHARED((NT * NB,), jnp.int32),
            pltpu.VMEM_SHARED((NT * 32,), jnp.float32),
            pltpu.SemaphoreType.DMA,
        ],
    )
    return f(loss)


@jax.jit
def kernel(outputs, target):
    loss = _ce_stage(outputs, target)
    res = _select_stage(loss.reshape(N))
    return res[0]


# speedup vs baseline: 1.0032x; 1.0032x over previous
import jax
import jax.numpy as jnp
from jax import lax
from jax.experimental import pallas as pl
from jax.experimental.pallas import tpu as pltpu
from jax.experimental.pallas import tpu_sc as plsc

MIN_KEPT = 100000
L07_BITS = 0x3EB69E19
C = 19
B = 4
H = 512
W = 512
N = B * H * W
ROWS = 64

NT = 16
CHUNK = N // NT
NVREG = CHUNK // 16
NB = 1024
HROW = NB + 16
KSEL = N - MIN_KEPT


def _ce_block(out_ref, tgt_ref, loss_ref):
    x = out_ref[0]
    t = tgt_ref[0]
    m = jnp.max(x, axis=0)
    e = jnp.exp(x - m[None])
    s = jnp.sum(e, axis=0)
    cls = lax.broadcasted_iota(jnp.int32, (C, ROWS, W), 0)
    xt = jnp.sum(jnp.where(cls == t[None], x, 0.0), axis=0)
    loss_ref[0] = jnp.log(s) - (xt - m)


def _ce_stage(outputs, target):
    grid = (B, H // ROWS)
    return pl.pallas_call(
        _ce_block,
        grid=grid,
        in_specs=[
            pl.BlockSpec((1, C, ROWS, W), lambda b, r: (b, 0, r, 0)),
            pl.BlockSpec((1, ROWS, W), lambda b, r: (b, r, 0)),
        ],
        out_specs=pl.BlockSpec((1, ROWS, W), lambda b, r: (b, r, 0)),
        out_shape=jax.ShapeDtypeStruct((B, H, W), jnp.float32),
    )(outputs, target)


def _sc_select_body(loss_hbm, out_hbm,
                    loss_v, hist_v, lred_v, hs_v, sums_v, su_v, out_v,
                    hist_sh, sums_sh, semp):
    w = lax.axis_index("s")
    lane = lax.broadcasted_iota(jnp.int32, (16,), 0)
    ones = jnp.full((16,), 1, jnp.int32)
    zeros16 = jnp.zeros((16,), jnp.int32)

    with jax.named_scope("sc_stage_dma"):
        pltpu.async_copy(
            loss_hbm.at[pl.ds(w * CHUNK, CHUNK)], loss_v, semp).wait()

    prefix = jnp.int32(0)
    kkeep = jnp.int32(KSEL)

    for sh_part, sh_buck in [(30, 20), (20, 10), (10, 0)]:
        def hz(j, _):
            for u in range(8):
                hist_v[pl.ds((j * 8 + u) * 16, 16)] = zeros16
            return 0
        with jax.named_scope("sc_hz"):
            lax.fori_loop(0, HROW * 16 // 128, hz, 0)

        pref_hi = prefix >> sh_part
        lskew = lane * (HROW + 1)

        def hist_step(j, _):
            for u in range(8):
                bits = plsc.bitcast(
                    loss_v[pl.ds((j * 8 + u) * 16, 16)], jnp.int32)
                part = (bits >> sh_part) == pref_hi
                bucket = (bits >> sh_buck) & (NB - 1)
                plsc.addupdate_scatter(
                    hist_v, [lskew + bucket], ones, mask=part)
            return 0
        with jax.named_scope("sc_hist"):
            lax.fori_loop(0, NVREG // 8, hist_step, 0)

        def lred_step(j, _):
            for u in range(4):
                b0 = (j * 4 + u) * 16
                acc = hist_v[pl.ds(b0, 16)]
                for l in range(1, 16):
                    acc = acc + hist_v[pl.ds(l * (HROW + 1) + b0, 16)]
                lred_v[pl.ds(b0, 16)] = acc
            return 0
        with jax.named_scope("sc_lred"):
            lax.fori_loop(0, 16, lred_step, 0)

        with jax.named_scope("sc_slots"):
            pltpu.sync_copy(lred_v, hist_sh.at[pl.ds(w * NB, NB)])
            plsc.subcore_barrier()

            pltpu.sync_copy(hist_sh, hs_v)

        def scan_step(j, carry):
            tot, accb, accc = carry
            h = hs_v[pl.ds(j * 16, 16)]
            for t in range(1, NT):
                h = h + hs_v[pl.ds(t * NB + j * 16, 16)]
            c = plsc.cumsum(h) + tot
            m = c < kkeep
            accb = accb + jnp.where(m, 1, 0)
            accc = accc + jnp.where(m, h, 0)
            return tot + jnp.sum(h), accb, accc
        with jax.named_scope("sc_scan"):
            _, accb, accc = lax.fori_loop(
                0, 64, scan_step, (jnp.int32(0), zeros16, zeros16))
        b_star = jnp.sum(accb)
        cnt_before = jnp.sum(accc)
        prefix = prefix | (b_star << sh_buck)
        kkeep = kkeep - cnt_before
        plsc.subcore_barrier()

    thr = jnp.minimum(prefix, jnp.int32(L07_BITS))

    def fin_step(j, carry):
        sa, ca = carry
        for u in range(8):
            lo = loss_v[pl.ds((j * 8 + u) * 16, 16)]
            m = plsc.bitcast(lo, jnp.int32) > thr
            sa = sa + jnp.where(m, lo, 0.0)
            ca = ca + jnp.where(m, 1.0, 0.0)
        return sa, ca
    with jax.named_scope("sc_final"):
        s_acc, c_acc = lax.fori_loop(
            0, NVREG // 8, fin_step,
            (jnp.zeros((16,), jnp.float32), jnp.zeros((16,), jnp.float32)))

    sums_v[pl.ds(0, 16)] = s_acc
    sums_v[pl.ds(16, 16)] = c_acc
    pltpu.sync_copy(sums_v, sums_sh.at[pl.ds(32 * w, 32)])
    plsc.subcore_barrier()

    @pl.when(w == 0)
    def _():
        pltpu.sync_copy(sums_sh, su_v)

        def red_step(t, carry):
            sv, cv = carry
            return (sv + su_v[pl.ds(t * 32, 16)],
                    cv + su_v[pl.ds(t * 32 + 16, 16)])
        sv, cv = lax.fori_loop(
            0, NT, red_step,
            (jnp.zeros((16,), jnp.float32), jnp.zeros((16,), jnp.float32)))
        tvec = jnp.full((16,), jnp.sum(sv), jnp.float32)
        dvec = jnp.maximum(jnp.full((16,), jnp.sum(cv), jnp.float32), 1.0)
        out_v[...] = tvec / dvec
        pltpu.sync_copy(out_v, out_hbm)


def _select_stage(loss):
    mesh = plsc.VectorSubcoreMesh(
        core_axis_name="c", subcore_axis_name="s", num_cores=1)
    f = pl.kernel(
        _sc_select_body,
        out_type=jax.ShapeDtypeStruct((16,), jnp.float32),
        mesh=mesh,
        compiler_params=pltpu.CompilerParams(needs_layout_passes=False),
        scratch_types=[
            pltpu.VMEM((CHUNK,), jnp.float32),
            pltpu.VMEM((16 * HROW + 16,), jnp.int32),
            pltpu.VMEM((NB,), jnp.int32),
            pltpu.VMEM((NT * NB,), jnp.int32),
            pltpu.VMEM((32,), jnp.float32),
            pltpu.VMEM((NT * 32,), jnp.float32),
            pltpu.VMEM((16,), jnp.float32),
            pltpu.VMEM_SHARED((NT * NB,), jnp.int32),
            pltpu.VMEM_SHARED((NT * 32,), jnp.float32),
            pltpu.SemaphoreType.DMA,
        ],
    )
    return f(loss)


@jax.jit
def kernel(outputs, target):
    loss = _ce_stage(outputs, target)
    res = _select_stage(loss.reshape(N))
    return res[0]


# speedup vs baseline: 1.5409x; 1.5359x over previous
import jax
import jax.numpy as jnp
from jax import lax
from jax.experimental import pallas as pl
from jax.experimental.pallas import tpu as pltpu
from jax.experimental.pallas import tpu_sc as plsc

MIN_KEPT = 100000
L07_BITS = 0x3EB69E19
C = 19
B = 4
H = 512
W = 512
N = B * H * W
ROWS = 64

NT = 16
CHUNK = N // NT
NVREG = CHUNK // 16
NB0 = 2048
NB = 1024
HROW = NB0 + 16
KSEL = N - MIN_KEPT


def _ce_block(out_ref, tgt_ref, loss_ref):
    x = out_ref[0]
    t = tgt_ref[0]
    m = jnp.max(x, axis=0)
    e = jnp.exp(x - m[None])
    s = jnp.sum(e, axis=0)
    cls = lax.broadcasted_iota(jnp.int32, (C, ROWS, W), 0)
    xt = jnp.sum(jnp.where(cls == t[None], x, 0.0), axis=0)
    loss_ref[0] = jnp.log(s) - (xt - m)


def _ce_stage(outputs, target):
    grid = (B, H // ROWS)
    return pl.pallas_call(
        _ce_block,
        grid=grid,
        in_specs=[
            pl.BlockSpec((1, C, ROWS, W), lambda b, r: (b, 0, r, 0)),
            pl.BlockSpec((1, ROWS, W), lambda b, r: (b, r, 0)),
        ],
        out_specs=pl.BlockSpec((1, ROWS, W), lambda b, r: (b, r, 0)),
        out_shape=jax.ShapeDtypeStruct((B, H, W), jnp.float32),
    )(outputs, target)


def _sc_select_body(loss_hbm, out_hbm,
                    loss_v, hist_v, lred_v, sums_v, su_v, out_v,
                    hist_sh, sums_sh, semp):
    w = lax.axis_index("s")
    lane = lax.broadcasted_iota(jnp.int32, (16,), 0)
    ones = jnp.full((16,), 1, jnp.int32)
    zeros16 = jnp.zeros((16,), jnp.int32)

    with jax.named_scope("sc_stage_dma"):
        pltpu.async_copy(
            loss_hbm.at[pl.ds(w * CHUNK, CHUNK)], loss_v, semp).wait()

    prefix = jnp.int32(0)
    kkeep = jnp.int32(KSEL)

    for sh_part, sh_buck, nb in [(31, 20, NB0), (20, 10, NB), (10, 0, NB)]:
        with jax.named_scope("sc_hz"):
            @plsc.parallel_loop(0, (nb + 16) * 16 // 16)
            def _(j):
                hist_v[pl.ds(j * 16, 16)] = zeros16

        pref_hi = prefix >> sh_part
        lskew = lane * (nb + 17)

        with jax.named_scope("sc_hist"):
            @plsc.parallel_loop(0, NVREG, unroll=8)
            def _(j):
                bits = plsc.bitcast(loss_v[pl.ds(j * 16, 16)], jnp.int32)
                part = (bits >> sh_part) == pref_hi
                bucket = (bits >> sh_buck) & (nb - 1)
                plsc.addupdate_scatter(
                    hist_v, [lskew + bucket], ones, mask=part)

        with jax.named_scope("sc_lred"):
            @plsc.parallel_loop(0, nb // 16, unroll=4)
            def _(j):
                b0 = j * 16
                acc = hist_v[pl.ds(b0, 16)]
                for l in range(1, 16):
                    acc = acc + hist_v[pl.ds(l * (nb + 17) + b0, 16)]
                lred_v[pl.ds(b0, 16)] = acc

        with jax.named_scope("sc_slots"):
            pltpu.sync_copy(lred_v.at[pl.ds(0, nb)],
                            hist_sh.at[pl.ds(w * nb, nb)])
            plsc.subcore_barrier()
            pltpu.sync_copy(hist_sh.at[pl.ds(0, NT * nb)],
                            hist_v.at[pl.ds(0, NT * nb)])

        def scan_step(j, carry):
            tot, accb, accc = carry
            h = hist_v[pl.ds(j * 16, 16)]
            for t in range(1, NT):
                h = h + hist_v[pl.ds(t * nb + j * 16, 16)]
            c = plsc.cumsum(h) + tot
            m = c < kkeep
            accb = accb + jnp.where(m, 1, 0)
            accc = accc + jnp.where(m, h, 0)
            return tot + jnp.sum(h), accb, accc
        with jax.named_scope("sc_scan"):
            _, accb, accc = lax.fori_loop(
                0, nb // 16, scan_step, (jnp.int32(0), zeros16, zeros16))
        b_star = jnp.sum(accb)
        cnt_before = jnp.sum(accc)
        prefix = prefix | (b_star << sh_buck)
        kkeep = kkeep - cnt_before
        plsc.subcore_barrier()

    thr = jnp.minimum(prefix, jnp.int32(L07_BITS))

    with jax.named_scope("sc_final"):
        @plsc.parallel_loop(
            0, NVREG, unroll=8,
            carry=(jnp.zeros((16,), jnp.float32),
                   jnp.zeros((16,), jnp.float32)))
        def fin_acc(j, carry):
            sa, ca = carry
            lo = loss_v[pl.ds(j * 16, 16)]
            m = plsc.bitcast(lo, jnp.int32) > thr
            return sa + jnp.where(m, lo, 0.0), ca + jnp.where(m, 1.0, 0.0)
        s_acc, c_acc = fin_acc

    sums_v[pl.ds(0, 16)] = s_acc
    sums_v[pl.ds(16, 16)] = c_acc
    pltpu.sync_copy(sums_v, sums_sh.at[pl.ds(32 * w, 32)])
    plsc.subcore_barrier()

    @pl.when(w == 0)
    def _():
        pltpu.sync_copy(sums_sh, su_v)

        def red_step(t, carry):
            sv, cv = carry
            return (sv + su_v[pl.ds(t * 32, 16)],
                    cv + su_v[pl.ds(t * 32 + 16, 16)])
        sv, cv = lax.fori_loop(
            0, NT, red_step,
            (jnp.zeros((16,), jnp.float32), jnp.zeros((16,), jnp.float32)))
        tvec = jnp.full((16,), jnp.sum(sv), jnp.float32)
        dvec = jnp.maximum(jnp.full((16,), jnp.sum(cv), jnp.float32), 1.0)
        out_v[...] = tvec / dvec
        pltpu.sync_copy(out_v, out_hbm)


def _select_stage(loss):
    mesh = plsc.VectorSubcoreMesh(
        core_axis_name="c", subcore_axis_name="s", num_cores=1)
    f = pl.kernel(
        _sc_select_body,
        out_type=jax.ShapeDtypeStruct((16,), jnp.float32),
        mesh=mesh,
        compiler_params=pltpu.CompilerParams(needs_layout_passes=False),
        scratch_types=[
            pltpu.VMEM((CHUNK,), jnp.float32),
            pltpu.VMEM((16 * HROW + 16,), jnp.int32),
            pltpu.VMEM((NB0,), jnp.int32),
            pltpu.VMEM((32,), jnp.float32),
            pltpu.VMEM((NT * 32,), jnp.float32),
            pltpu.VMEM((16,), jnp.float32),
            pltpu.VMEM_SHARED((NT * NB0,), jnp.int32),
            pltpu.VMEM_SHARED((NT * 32,), jnp.float32),
            pltpu.SemaphoreType.DMA,
        ],
    )
    return f(loss)


@jax.jit
def kernel(outputs, target):
    loss = _ce_stage(outputs, target)
    res = _select_stage(loss.reshape(N))
    return res[0]


# speedup vs baseline: 1.7993x; 1.1677x over previous
import jax
import jax.numpy as jnp
from jax import lax
from jax.experimental import pallas as pl
from jax.experimental.pallas import tpu as pltpu
from jax.experimental.pallas import tpu_sc as plsc

MIN_KEPT = 100000
L07_BITS = 0x3EB69E19
C = 19
B = 4
H = 512
W = 512
N = B * H * W
ROWS = 64

NT = 16
CHUNK = N // NT
NVREG = CHUNK // 16
NB0 = 2048
NB = 1024
HROW = NB0 + 16
KSEL = N - MIN_KEPT


def _ce_block(out_ref, tgt_ref, loss_ref):
    x = out_ref[0]
    t = tgt_ref[0]
    m = jnp.max(x, axis=0)
    e = jnp.exp(x - m[None])
    s = jnp.sum(e, axis=0)
    cls = lax.broadcasted_iota(jnp.int32, (C, ROWS, W), 0)
    xt = jnp.sum(jnp.where(cls == t[None], x, 0.0), axis=0)
    loss_ref[0] = jnp.log(s) - (xt - m)


def _ce_stage(outputs, target):
    grid = (B, H // ROWS)
    return pl.pallas_call(
        _ce_block,
        grid=grid,
        in_specs=[
            pl.BlockSpec((1, C, ROWS, W), lambda b, r: (b, 0, r, 0)),
            pl.BlockSpec((1, ROWS, W), lambda b, r: (b, r, 0)),
        ],
        out_specs=pl.BlockSpec((1, ROWS, W), lambda b, r: (b, r, 0)),
        out_shape=jax.ShapeDtypeStruct((B, H, W), jnp.float32),
    )(outputs, target)


def _sc_select_body(loss_hbm, out_hbm,
                    loss_v, hist_v, lred_v, sums_v, su_v, out_v,
                    hist_sh, sums_sh, semp):
    w = lax.axis_index("s")
    lane = lax.broadcasted_iota(jnp.int32, (16,), 0)
    ones = jnp.full((16,), 1, jnp.int32)
    zeros16 = jnp.zeros((16,), jnp.int32)

    with jax.named_scope("sc_stage_dma"):
        pltpu.async_copy(
            loss_hbm.at[pl.ds(w * CHUNK, CHUNK)], loss_v, semp).wait()

    prefix = jnp.int32(0)
    kkeep = jnp.int32(KSEL)

    for sh_part, sh_buck, nb in [(31, 20, NB0), (20, 10, NB), (10, 0, NB)]:
        with jax.named_scope("sc_hz"):
            @plsc.parallel_loop(0, (nb + 16) * 16 // 16, unroll=8)
            def _(j):
                hist_v[pl.ds(j * 16, 16)] = zeros16

        pref_hi = prefix >> sh_part
        lskew = lane * (nb + 17)

        with jax.named_scope("sc_hist"):
            @plsc.parallel_loop(0, NVREG, unroll=8)
            def _(j):
                bits = plsc.bitcast(loss_v[pl.ds(j * 16, 16)], jnp.int32)
                part = (bits >> sh_part) == pref_hi
                bucket = (bits >> sh_buck) & (nb - 1)
                plsc.addupdate_scatter(
                    hist_v, [lskew + bucket], ones, mask=part)

        with jax.named_scope("sc_lred"):
            @plsc.parallel_loop(0, nb // 16, unroll=4)
            def _(j):
                b0 = j * 16
                acc = hist_v[pl.ds(b0, 16)]
                for l in range(1, 16):
                    acc = acc + hist_v[pl.ds(l * (nb + 17) + b0, 16)]
                lred_v[pl.ds(b0, 16)] = acc

        with jax.named_scope("sc_slots"):
            pltpu.sync_copy(lred_v.at[pl.ds(0, nb)],
                            hist_sh.at[pl.ds(w * nb, nb)])
            plsc.subcore_barrier()
            pltpu.sync_copy(hist_sh.at[pl.ds(0, NT * nb)],
                            hist_v.at[pl.ds(0, NT * nb)])

        def scan_step(j, carry):
            tot, accb, accc = carry
            h = hist_v[pl.ds(j * 16, 16)]
            for t in range(1, NT):
                h = h + hist_v[pl.ds(t * nb + j * 16, 16)]
            c = plsc.cumsum(h) + tot
            m = c < kkeep
            accb = accb + jnp.where(m, 1, 0)
            accc = accc + jnp.where(m, h, 0)
            return tot + jnp.sum(h), accb, accc
        with jax.named_scope("sc_scan"):
            _, accb, accc = lax.fori_loop(
                0, nb // 16, scan_step, (jnp.int32(0), zeros16, zeros16))
        b_star = jnp.sum(accb)
        cnt_before = jnp.sum(accc)
        prefix = prefix | (b_star << sh_buck)
        kkeep = kkeep - cnt_before
        plsc.subcore_barrier()

    thr = jnp.minimum(prefix, jnp.int32(L07_BITS))

    with jax.named_scope("sc_final"):
        @plsc.parallel_loop(
            0, NVREG, unroll=8,
            carry=(jnp.zeros((16,), jnp.float32),
                   jnp.zeros((16,), jnp.float32)))
        def fin_acc(j, carry):
            sa, ca = carry
            lo = loss_v[pl.ds(j * 16, 16)]
            m = plsc.bitcast(lo, jnp.int32) > thr
            return sa + jnp.where(m, lo, 0.0), ca + jnp.where(m, 1.0, 0.0)
        s_acc, c_acc = fin_acc

    sums_v[pl.ds(0, 16)] = s_acc
    sums_v[pl.ds(16, 16)] = c_acc
    pltpu.sync_copy(sums_v, sums_sh.at[pl.ds(32 * w, 32)])
    plsc.subcore_barrier()

    @pl.when(w == 0)
    def _():
        pltpu.sync_copy(sums_sh, su_v)

        def red_step(t, carry):
            sv, cv = carry
            return (sv + su_v[pl.ds(t * 32, 16)],
                    cv + su_v[pl.ds(t * 32 + 16, 16)])
        sv, cv = lax.fori_loop(
            0, NT, red_step,
            (jnp.zeros((16,), jnp.float32), jnp.zeros((16,), jnp.float32)))
        tvec = jnp.full((16,), jnp.sum(sv), jnp.float32)
        dvec = jnp.maximum(jnp.full((16,), jnp.sum(cv), jnp.float32), 1.0)
        out_v[...] = tvec / dvec
        pltpu.sync_copy(out_v, out_hbm)


def _select_stage(loss):
    mesh = plsc.VectorSubcoreMesh(
        core_axis_name="c", subcore_axis_name="s", num_cores=1)
    f = pl.kernel(
        _sc_select_body,
        out_type=jax.ShapeDtypeStruct((16,), jnp.float32),
        mesh=mesh,
        compiler_params=pltpu.CompilerParams(needs_layout_passes=False),
        scratch_types=[
            pltpu.VMEM((CHUNK,), jnp.float32),
            pltpu.VMEM((16 * HROW + 16,), jnp.int32),
            pltpu.VMEM((NB0,), jnp.int32),
            pltpu.VMEM((32,), jnp.float32),
            pltpu.VMEM((NT * 32,), jnp.float32),
            pltpu.VMEM((16,), jnp.float32),
            pltpu.VMEM_SHARED((NT * NB0,), jnp.int32),
            pltpu.VMEM_SHARED((NT * 32,), jnp.float32),
            pltpu.SemaphoreType.DMA,
        ],
    )
    return f(loss)


@jax.jit
def kernel(outputs, target):
    loss = _ce_stage(outputs, target)
    res = _select_stage(loss.reshape(N))
    return res[0]


# speedup vs baseline: 2.0078x; 1.1159x over previous
import jax
import jax.numpy as jnp
from jax import lax
from jax.experimental import pallas as pl
from jax.experimental.pallas import tpu as pltpu
from jax.experimental.pallas import tpu_sc as plsc

MIN_KEPT = 100000
L07_BITS = 0x3EB69E19
C = 19
B = 4
H = 512
W = 512
N = B * H * W
ROWS = 128

NT = 16
CHUNK = N // NT
NVREG = CHUNK // 16
NB0 = 2048
NB = 1024
HROW = NB0 + 16
KSEL = N - MIN_KEPT


def _ce_block(out_ref, tgt_ref, loss_ref):
    x = out_ref[0]
    t = tgt_ref[0]
    m = jnp.max(x, axis=0)
    e = jnp.exp(x - m[None])
    s = jnp.sum(e, axis=0)
    cls = lax.broadcasted_iota(jnp.int32, (C, ROWS, W), 0)
    xt = jnp.sum(jnp.where(cls == t[None], x, 0.0), axis=0)
    loss_ref[0] = jnp.log(s) - (xt - m)


def _ce_stage(outputs, target):
    grid = (B, H // ROWS)
    return pl.pallas_call(
        _ce_block,
        grid=grid,
        in_specs=[
            pl.BlockSpec((1, C, ROWS, W), lambda b, r: (b, 0, r, 0)),
            pl.BlockSpec((1, ROWS, W), lambda b, r: (b, r, 0)),
        ],
        out_specs=pl.BlockSpec((1, ROWS, W), lambda b, r: (b, r, 0)),
        out_shape=jax.ShapeDtypeStruct((B, H, W), jnp.float32),
    )(outputs, target)


def _sc_select_body(loss_hbm, out_hbm,
                    loss_v, hist_v, lred_v, sums_v, su_v, out_v,
                    hist_sh, sums_sh, semp):
    w = lax.axis_index("s")
    lane = lax.broadcasted_iota(jnp.int32, (16,), 0)
    ones = jnp.full((16,), 1, jnp.int32)
    zeros16 = jnp.zeros((16,), jnp.int32)

    with jax.named_scope("sc_stage_dma"):
        cp = pltpu.async_copy(
            loss_hbm.at[pl.ds(w * CHUNK, CHUNK)], loss_v, semp)

    prefix = jnp.int32(0)
    kkeep = jnp.int32(KSEL)

    for lvl, (sh_part, sh_buck, nb) in enumerate(
            [(31, 20, NB0), (20, 10, NB), (10, 0, NB)]):
        with jax.named_scope("sc_hz"):
            @plsc.parallel_loop(0, (nb + 16) * 16 // 16, unroll=8)
            def _(j):
                hist_v[pl.ds(j * 16, 16)] = zeros16
        if lvl == 0:
            with jax.named_scope("sc_stage_dma"):
                cp.wait()

        pref_hi = prefix >> sh_part
        lskew = lane * (nb + 17)

        with jax.named_scope("sc_hist"):
            @plsc.parallel_loop(0, NVREG, unroll=8)
            def _(j):
                bits = plsc.bitcast(loss_v[pl.ds(j * 16, 16)], jnp.int32)
                part = (bits >> sh_part) == pref_hi
                bucket = (bits >> sh_buck) & (nb - 1)
                plsc.addupdate_scatter(
                    hist_v, [lskew + bucket], ones, mask=part)

        with jax.named_scope("sc_lred"):
            @plsc.parallel_loop(0, nb // 16, unroll=4)
            def _(j):
                b0 = j * 16
                acc = hist_v[pl.ds(b0, 16)]
                for l in range(1, 16):
                    acc = acc + hist_v[pl.ds(l * (nb + 17) + b0, 16)]
                lred_v[pl.ds(b0, 16)] = acc

        with jax.named_scope("sc_slots"):
            pltpu.sync_copy(lred_v.at[pl.ds(0, nb)],
                            hist_sh.at[pl.ds(w * nb, nb)])
            plsc.subcore_barrier()
            pltpu.sync_copy(hist_sh.at[pl.ds(0, NT * nb)],
                            hist_v.at[pl.ds(0, NT * nb)])

        def scan_step(j, carry):
            tot, accb, accc = carry
            h = hist_v[pl.ds(j * 16, 16)]
            for t in range(1, NT):
                h = h + hist_v[pl.ds(t * nb + j * 16, 16)]
            c = plsc.cumsum(h) + tot
            m = c < kkeep
            accb = accb + jnp.where(m, 1, 0)
            accc = accc + jnp.where(m, h, 0)
            return tot + jnp.sum(h), accb, accc
        with jax.named_scope("sc_scan"):
            _, accb, accc = lax.fori_loop(
                0, nb // 16, scan_step, (jnp.int32(0), zeros16, zeros16))
        b_star = jnp.sum(accb)
        cnt_before = jnp.sum(accc)
        prefix = prefix | (b_star << sh_buck)
        kkeep = kkeep - cnt_before
        plsc.subcore_barrier()

    thr = jnp.minimum(prefix, jnp.int32(L07_BITS))

    zf = jnp.zeros((16,), jnp.float32)
    with jax.named_scope("sc_final"):
        @plsc.parallel_loop(
            0, NVREG // 4, unroll=4, carry=((zf, zf), (zf, zf),
                                            (zf, zf), (zf, zf)))
        def fin_acc(j, carry):
            out = []
            for q in range(4):
                sa, ca = carry[q]
                lo = loss_v[pl.ds((4 * j + q) * 16, 16)]
                m = plsc.bitcast(lo, jnp.int32) > thr
                out.append((sa + jnp.where(m, lo, 0.0),
                            ca + jnp.where(m, 1.0, 0.0)))
            return tuple(out)
        (s0, c0), (s1, c1), (s2, c2), (s3, c3) = fin_acc
        s_acc = (s0 + s1) + (s2 + s3)
        c_acc = (c0 + c1) + (c2 + c3)

    sums_v[pl.ds(0, 16)] = s_acc
    sums_v[pl.ds(16, 16)] = c_acc
    pltpu.sync_copy(sums_v, sums_sh.at[pl.ds(32 * w, 32)])
    plsc.subcore_barrier()

    @pl.when(w == 0)
    def _():
        pltpu.sync_copy(sums_sh, su_v)

        def red_step(t, carry):
            sv, cv = carry
            return (sv + su_v[pl.ds(t * 32, 16)],
                    cv + su_v[pl.ds(t * 32 + 16, 16)])
        sv, cv = lax.fori_loop(
            0, NT, red_step,
            (jnp.zeros((16,), jnp.float32), jnp.zeros((16,), jnp.float32)))
        tvec = jnp.full((16,), jnp.sum(sv), jnp.float32)
        dvec = jnp.maximum(jnp.full((16,), jnp.sum(cv), jnp.float32), 1.0)
        out_v[...] = tvec / dvec
        pltpu.sync_copy(out_v, out_hbm)


def _select_stage(loss):
    mesh = plsc.VectorSubcoreMesh(
        core_axis_name="c", subcore_axis_name="s", num_cores=1)
    f = pl.kernel(
        _sc_select_body,
        out_type=jax.ShapeDtypeStruct((16,), jnp.float32),
        mesh=mesh,
        compiler_params=pltpu.CompilerParams(needs_layout_passes=False),
        scratch_types=[
            pltpu.VMEM((CHUNK,), jnp.float32),
            pltpu.VMEM((16 * HROW + 16,), jnp.int32),
            pltpu.VMEM((NB0,), jnp.int32),
            pltpu.VMEM((32,), jnp.float32),
            pltpu.VMEM((NT * 32,), jnp.float32),
            pltpu.VMEM((16,), jnp.float32),
            pltpu.VMEM_SHARED((NT * NB0,), jnp.int32),
            pltpu.VMEM_SHARED((NT * 32,), jnp.float32),
            pltpu.SemaphoreType.DMA,
        ],
    )
    return f(loss)


@jax.jit
def kernel(outputs, target):
    loss = _ce_stage(outputs, target)
    res = _select_stage(loss.reshape(N))
    return res[0]


# speedup vs baseline: 2.0785x; 1.0352x over previous
import jax
import jax.numpy as jnp
from jax import lax
from jax.experimental import pallas as pl
from jax.experimental.pallas import tpu as pltpu
from jax.experimental.pallas import tpu_sc as plsc

MIN_KEPT = 100000
L07_BITS = 0x3EB69E19
C = 19
B = 4
H = 512
W = 512
N = B * H * W
ROWS = 256

NT = 16
CHUNK = N // NT
NVREG = CHUNK // 16
NB0 = 2048
NB = 1024
HROW = NB0 + 16
KSEL = N - MIN_KEPT


def _ce_block(out_ref, tgt_ref, loss_ref):
    x = out_ref[0]
    t = tgt_ref[0]
    m = jnp.max(x, axis=0)
    e = jnp.exp(x - m[None])
    s = jnp.sum(e, axis=0)
    cls = lax.broadcasted_iota(jnp.int32, (C, ROWS, W), 0)
    xt = jnp.sum(jnp.where(cls == t[None], x, 0.0), axis=0)
    loss_ref[0] = jnp.log(s) - (xt - m)


def _ce_stage(outputs, target):
    grid = (B, H // ROWS)
    return pl.pallas_call(
        _ce_block,
        grid=grid,
        in_specs=[
            pl.BlockSpec((1, C, ROWS, W), lambda b, r: (b, 0, r, 0)),
            pl.BlockSpec((1, ROWS, W), lambda b, r: (b, r, 0)),
        ],
        out_specs=pl.BlockSpec((1, ROWS, W), lambda b, r: (b, r, 0)),
        out_shape=jax.ShapeDtypeStruct((B, H, W), jnp.float32),
    )(outputs, target)


def _sc_select_body(loss_hbm, out_hbm,
                    loss_v, hist_v, lred_v, sums_v, su_v, out_v,
                    hist_sh, sums_sh, semp):
    w = lax.axis_index("s")
    lane = lax.broadcasted_iota(jnp.int32, (16,), 0)
    ones = jnp.full((16,), 1, jnp.int32)
    zeros16 = jnp.zeros((16,), jnp.int32)

    with jax.named_scope("sc_stage_dma"):
        cp = pltpu.async_copy(
            loss_hbm.at[pl.ds(w * CHUNK, CHUNK)], loss_v, semp)

    prefix = jnp.int32(0)
    kkeep = jnp.int32(KSEL)

    for lvl, (sh_part, sh_buck, nb) in enumerate(
            [(31, 20, NB0), (20, 10, NB), (10, 0, NB)]):
        with jax.named_scope("sc_hz"):
            @plsc.parallel_loop(0, (nb + 16) * 16 // 16, unroll=8)
            def _(j):
                hist_v[pl.ds(j * 16, 16)] = zeros16
        if lvl == 0:
            with jax.named_scope("sc_stage_dma"):
                cp.wait()

        pref_hi = prefix >> sh_part
        lskew = lane * (nb + 17)

        with jax.named_scope("sc_hist"):
            @plsc.parallel_loop(0, NVREG, unroll=8)
            def _(j):
                bits = plsc.bitcast(loss_v[pl.ds(j * 16, 16)], jnp.int32)
                part = (bits >> sh_part) == pref_hi
                bucket = (bits >> sh_buck) & (nb - 1)
                plsc.addupdate_scatter(
                    hist_v, [lskew + bucket], ones, mask=part)

        with jax.named_scope("sc_lred"):
            @plsc.parallel_loop(0, nb // 16, unroll=4)
            def _(j):
                b0 = j * 16
                acc = hist_v[pl.ds(b0, 16)]
                for l in range(1, 16):
                    acc = acc + hist_v[pl.ds(l * (nb + 17) + b0, 16)]
                lred_v[pl.ds(b0, 16)] = acc

        with jax.named_scope("sc_slots"):
            pltpu.sync_copy(lred_v.at[pl.ds(0, nb)],
                            hist_sh.at[pl.ds(w * nb, nb)])
            plsc.subcore_barrier()
            pltpu.sync_copy(hist_sh.at[pl.ds(0, NT * nb)],
                            hist_v.at[pl.ds(0, NT * nb)])

        def scan_step(j, carry):
            tot, accb, accc = carry
            h = hist_v[pl.ds(j * 16, 16)]
            for t in range(1, NT):
                h = h + hist_v[pl.ds(t * nb + j * 16, 16)]
            c = plsc.cumsum(h) + tot
            m = c < kkeep
            accb = accb + jnp.where(m, 1, 0)
            accc = accc + jnp.where(m, h, 0)
            return tot + jnp.sum(h), accb, accc
        with jax.named_scope("sc_scan"):
            _, accb, accc = lax.fori_loop(
                0, nb // 16, scan_step, (jnp.int32(0), zeros16, zeros16))
        b_star = jnp.sum(accb)
        cnt_before = jnp.sum(accc)
        prefix = prefix | (b_star << sh_buck)
        kkeep = kkeep - cnt_before
        plsc.subcore_barrier()

    thr = jnp.minimum(prefix, jnp.int32(L07_BITS))

    zf = jnp.zeros((16,), jnp.float32)
    with jax.named_scope("sc_final"):
        @plsc.parallel_loop(
            0, NVREG // 4, unroll=4, carry=((zf, zf), (zf, zf),
                                            (zf, zf), (zf, zf)))
        def fin_acc(j, carry):
            out = []
            for q in range(4):
                sa, ca = carry[q]
                lo = loss_v[pl.ds((4 * j + q) * 16, 16)]
                m = plsc.bitcast(lo, jnp.int32) > thr
                out.append((sa + jnp.where(m, lo, 0.0),
                            ca + jnp.where(m, 1.0, 0.0)))
            return tuple(out)
        (s0, c0), (s1, c1), (s2, c2), (s3, c3) = fin_acc
        s_acc = (s0 + s1) + (s2 + s3)
        c_acc = (c0 + c1) + (c2 + c3)

    sums_v[pl.ds(0, 16)] = s_acc
    sums_v[pl.ds(16, 16)] = c_acc
    pltpu.sync_copy(sums_v, sums_sh.at[pl.ds(32 * w, 32)])
    plsc.subcore_barrier()

    @pl.when(w == 0)
    def _():
        pltpu.sync_copy(sums_sh, su_v)

        def red_step(t, carry):
            sv, cv = carry
            return (sv + su_v[pl.ds(t * 32, 16)],
                    cv + su_v[pl.ds(t * 32 + 16, 16)])
        sv, cv = lax.fori_loop(
            0, NT, red_step,
            (jnp.zeros((16,), jnp.float32), jnp.zeros((16,), jnp.float32)))
        tvec = jnp.full((16,), jnp.sum(sv), jnp.float32)
        dvec = jnp.maximum(jnp.full((16,), jnp.sum(cv), jnp.float32), 1.0)
        out_v[...] = tvec / dvec
        pltpu.sync_copy(out_v, out_hbm)


def _select_stage(loss):
    mesh = plsc.VectorSubcoreMesh(
        core_axis_name="c", subcore_axis_name="s", num_cores=1)
    f = pl.kernel(
        _sc_select_body,
        out_type=jax.ShapeDtypeStruct((16,), jnp.float32),
        mesh=mesh,
        compiler_params=pltpu.CompilerParams(needs_layout_passes=False),
        scratch_types=[
            pltpu.VMEM((CHUNK,), jnp.float32),
            pltpu.VMEM((16 * HROW + 16,), jnp.int32),
            pltpu.VMEM((NB0,), jnp.int32),
            pltpu.VMEM((32,), jnp.float32),
            pltpu.VMEM((NT * 32,), jnp.float32),
            pltpu.VMEM((16,), jnp.float32),
            pltpu.VMEM_SHARED((NT * NB0,), jnp.int32),
            pltpu.VMEM_SHARED((NT * 32,), jnp.float32),
            pltpu.SemaphoreType.DMA,
        ],
    )
    return f(loss)


@jax.jit
def kernel(outputs, target):
    loss = _ce_stage(outputs, target)
    res = _select_stage(loss.reshape(N))
    return res[0]
